# baseline (speedup 1.0000x reference)
import functools

import jax
import jax.numpy as jnp
from jax import lax
from jax.experimental import pallas as pl
from jax.experimental.pallas import tpu as pltpu
from jax.experimental.pallas import tpu_sc as plsc

_EMBED = 64
_NC = 2
_NS = 16
_NW = _NC * _NS


@functools.lru_cache(maxsize=None)
def _make_gather(B: int):
    b_per_w = B // _NW
    C = 800
    n = b_per_w // C
    mesh = plsc.VectorSubcoreMesh(core_axis_name="c", subcore_axis_name="s")

    @functools.partial(
        pl.kernel,
        mesh=mesh,
        out_type=jax.ShapeDtypeStruct((B, _EMBED), jnp.float32),
        scratch_types=[
            pltpu.VMEM((C,), jnp.int32),
            pltpu.VMEM((C,), jnp.int32),
            pltpu.VMEM((C, _EMBED), jnp.float32),
            pltpu.VMEM((C, _EMBED), jnp.float32),
            pltpu.SemaphoreType.DMA,
            pltpu.SemaphoreType.DMA,
            pltpu.SemaphoreType.DMA,
            pltpu.SemaphoreType.DMA,
        ],
        compiler_params=pltpu.CompilerParams(use_tc_tiling_on_sc=False),
    )
    def gather(idx_hbm, table_hbm, out_hbm, i0, i1, r0, r1, sg0, sg1, st0, st1):
        wid = lax.axis_index("s") * _NC + lax.axis_index("c")
        base = wid * b_per_w
        idx_bufs = (i0, i1)
        row_bufs = (r0, r1)
        g_sems = (sg0, sg1)
        s_sems = (st0, st1)

        def idx_load(k, slot):
            pltpu.sync_copy(idx_hbm.at[pl.ds(base + k * C, C)], idx_bufs[slot])

        def gather_start(slot):
            pltpu.async_copy(table_hbm.at[idx_bufs[slot]], row_bufs[slot],
                             g_sems[slot])

        def gather_wait(slot):
            pltpu.make_async_copy(table_hbm.at[idx_bufs[slot]], row_bufs[slot],
                                  g_sems[slot]).wait()

        def store_start(k, slot):
            pltpu.async_copy(row_bufs[slot],
                             out_hbm.at[pl.ds(base + k * C, C)], s_sems[slot])

        def store_wait(k, slot):
            pltpu.make_async_copy(row_bufs[slot],
                                  out_hbm.at[pl.ds(base + k * C, C)],
                                  s_sems[slot]).wait()

        idx_load(0, 0)
        gather_start(0)
        idx_load(1, 1)
        gather_wait(0)
        store_start(0, 0)
        idx_load(2, 0)
        gather_start(1)

        def body(t, carry):
            j1 = 2 * t + 1
            j2 = j1 + 1
            gather_wait(1)
            store_start(j1, 1)
            idx_load(jnp.minimum(j1 + 2, n - 1), 1)
            store_wait(j1 - 1, 0)
            gather_start(0)
            gather_wait(0)
            store_start(j2, 0)
            idx_load(jnp.minimum(j2 + 2, n - 1), 0)
            store_wait(j2 - 1, 1)
            gather_start(1)
            return carry

        lax.fori_loop(0, (n - 2) // 2, body, 0)

        gather_wait(1)
        store_start(n - 1, 1)
        store_wait(n - 2, 0)
        store_wait(n - 1, 1)

    return gather


def kernel(x, table):
    B = x.shape[0] * x.shape[1]
    out = _make_gather(B)(x.reshape(B), table)
    return out.reshape(x.shape[0], x.shape[1], _EMBED)
